# baseline (speedup 1.0000x reference)
import functools

import numpy as np
import jax
import jax.numpy as jnp
from jax import lax
from jax.experimental import pallas as pl
from jax.experimental.pallas import tpu as pltpu
from jax.experimental.pallas import tpu_sc as plsc

_NC = 2
_NS = 16
_NW = _NC * _NS
_LANES = 16
_PCH = 8
_NSLOT = 3


def _pos_encoding_packed_np(length: int, d_model: int) -> np.ndarray:
    depth = d_model / 2
    depths = np.arange(depth)[np.newaxis, :] / depth
    angle_rads = np.arange(length)[:, np.newaxis] / 10000 ** depths
    pos = np.concatenate([np.sin(angle_rads), np.cos(angle_rads)], axis=-1)
    q = np.clip(np.rint(pos * 127.0), -127, 127).astype(np.int8)
    bits = q.view(np.uint8).astype(np.uint32)
    bits = bits.reshape(length, d_model // 64, 4, 16)
    words = (bits[:, :, 0, :] | (bits[:, :, 1, :] << 8)
             | (bits[:, :, 2, :] << 16) | (bits[:, :, 3, :] << 24))
    return words.reshape(length, d_model // 4).view(np.int32)


@jax.jit
def _run(x, table, pos):
    bsz, seq_len = x.shape
    vocab, d = table.shape
    n_tok = bsz * seq_len
    ppw = seq_len // _NW
    nch = ppw // _PCH
    rpc = bsz * _PCH
    ngrp = d // 64
    scale = float(np.sqrt(d))
    dq = 1.0 / 127.0

    idx = (x.astype(jnp.int32)
            .reshape(bsz, _NW, nch, _PCH)
            .transpose(1, 2, 0, 3)
            .reshape(n_tok))

    mesh = plsc.VectorSubcoreMesh(core_axis_name="c", subcore_axis_name="s")

    @functools.partial(
        pl.kernel,
        mesh=mesh,
        out_type=jax.ShapeDtypeStruct((n_tok, d), jnp.float32),
        scratch_types=[
            pltpu.VMEM((bsz * ppw,), jnp.int32),
            pltpu.VMEM((_NSLOT, rpc, d), jnp.float32),
            pltpu.VMEM((_NSLOT, _PCH, d // 4), jnp.int32),
            pltpu.SemaphoreType.DMA((_NSLOT,)),
            pltpu.SemaphoreType.DMA((_NSLOT,)),
            pltpu.SemaphoreType.DMA((_NSLOT,)),
            pltpu.SemaphoreType.DMA,
        ],
    )
    def emb(idx_hbm, tab_hbm, pos_hbm, out_hbm, idx2_v, rows_v, pbuf_v,
            gsem, psem, osem, isem):
        wid = lax.axis_index("s") * _NC + lax.axis_index("c")
        pbase = wid * ppw

        def start_pos(c):
            sl = c % _NSLOT
            return pltpu.async_copy(
                pos_hbm.at[pl.ds(pbase + c * _PCH, _PCH)],
                pbuf_v.at[sl], psem.at[sl])

        def start_gathers(c):
            sl = c % _NSLOT
            return [pltpu.async_copy(
                tab_hbm.at[idx2_v.at[pl.ds(c * rpc, rpc)]],
                rows_v.at[sl], gsem.at[sl])]

        def start_chunk(c):
            return [start_pos(c)] + start_gathers(c)

        pf = _NSLOT - 1
        inflight = {}
        outflight = {}

        pos_copies = {c: start_pos(c) for c in range(min(pf, nch))}
        pltpu.async_copy(
            idx_hbm.at[pl.ds(wid * bsz * ppw, bsz * ppw)], idx2_v,
            isem).wait()

        for c in range(min(pf, nch)):
            inflight[c] = [pos_copies[c]] + start_gathers(c)

        for c in range(nch):
            sl = c % _NSLOT
            for copy in inflight.pop(c):
                copy.wait()

            def body(t, carry):
                p = t // 4
                jq = t % 4
                for g in range(ngrp // 4):
                    colw = (jq * (ngrp // 4) + g) * _LANES
                    col = colw * 4
                    pw = pbuf_v[sl, p, pl.ds(colw, _LANES)]
                    for k in range(4):
                        shl = pw << (24 - 8 * k) if k < 3 else pw
                        pv = lax.convert_element_type(
                            lax.shift_right_arithmetic(shl, 24),
                            jnp.float32) * dq
                        ck = col + k * _LANES
                        for b in range(bsz):
                            r = b * _PCH + p
                            v = rows_v[sl, r, pl.ds(ck, _LANES)] * scale + pv
                            rows_v[sl, r, pl.ds(ck, _LANES)] = v
                return carry

            lax.fori_loop(0, _PCH * 4, body, 0)

            wcopies = []
            for b in range(bsz):
                wcopies.append(pltpu.async_copy(
                    rows_v.at[sl, pl.ds(b * _PCH, _PCH)],
                    out_hbm.at[pl.ds(b * seq_len + pbase + c * _PCH, _PCH)],
                    osem.at[sl]))
            outflight[c] = wcopies

            nxt = c + pf
            if nxt < nch:
                prev = nxt - _NSLOT
                if prev >= 0:
                    for copy in outflight.pop(prev):
                        copy.wait()
                inflight[nxt] = start_chunk(nxt)

        for c in sorted(outflight):
            for copy in outflight.pop(c):
                copy.wait()

    return emb(idx, table, pos)


def kernel(x, table):
    b, s = x.shape
    vocab, d = table.shape
    pos = jnp.asarray(_pos_encoding_packed_np(s, d))
    out = _run(x, table, pos)
    return out.reshape(b, s, d)


# speedup vs baseline: 1.0298x; 1.0298x over previous
import functools

import numpy as np
import jax
import jax.numpy as jnp
from jax import lax
from jax.experimental import pallas as pl
from jax.experimental.pallas import tpu as pltpu
from jax.experimental.pallas import tpu_sc as plsc

_NC = 2
_NS = 16
_NW = _NC * _NS
_LANES = 16
_PCH = 8
_NSLOT = 3


def _pos_encoding_packed_np(length: int, d_model: int) -> np.ndarray:
    depth = d_model / 2
    depths = np.arange(depth)[np.newaxis, :] / depth
    angle_rads = np.arange(length)[:, np.newaxis] / 10000 ** depths
    pos = np.concatenate([np.sin(angle_rads), np.cos(angle_rads)], axis=-1)
    q = np.clip(np.rint(pos * 127.0), -127, 127).astype(np.int8)
    bits = q.view(np.uint8).astype(np.uint32)
    bits = bits.reshape(length, d_model // 64, 4, 16)
    words = (bits[:, :, 0, :] | (bits[:, :, 1, :] << 8)
             | (bits[:, :, 2, :] << 16) | (bits[:, :, 3, :] << 24))
    return words.reshape(length, d_model // 4).view(np.int32)


@jax.jit
def _run(x, table, pos):
    bsz, seq_len = x.shape
    vocab, d = table.shape
    n_tok = bsz * seq_len
    ppw = seq_len // _NW
    nch = ppw // _PCH
    rpc = bsz * _PCH
    ngrp = d // 64
    scale = float(np.sqrt(d))
    dq = 1.0 / 127.0

    mesh = plsc.VectorSubcoreMesh(core_axis_name="c", subcore_axis_name="s")

    @functools.partial(
        pl.kernel,
        mesh=mesh,
        out_type=jax.ShapeDtypeStruct((n_tok, d), jnp.float32),
        scratch_types=[
            pltpu.VMEM((bsz, ppw), jnp.int32),
            pltpu.VMEM((_NSLOT, rpc, d), jnp.float32),
            pltpu.VMEM((_NSLOT, _PCH, d // 4), jnp.int32),
            pltpu.SemaphoreType.DMA((_NSLOT,)),
            pltpu.SemaphoreType.DMA((_NSLOT,)),
            pltpu.SemaphoreType.DMA((_NSLOT,)),
            pltpu.SemaphoreType.DMA,
        ],
    )
    def emb(x_hbm, tab_hbm, pos_hbm, out_hbm, idx_v, rows_v, pbuf_v,
            gsem, psem, osem, isem):
        wid = lax.axis_index("s") * _NC + lax.axis_index("c")
        pbase = wid * ppw

        def start_pos(c):
            sl = c % _NSLOT
            return pltpu.async_copy(
                pos_hbm.at[pl.ds(pbase + c * _PCH, _PCH)],
                pbuf_v.at[sl], psem.at[sl])

        def start_gathers(c):
            sl = c % _NSLOT
            return [pltpu.async_copy(
                tab_hbm.at[idx_v.at[b, pl.ds(c * _PCH, _PCH)]],
                rows_v.at[sl, pl.ds(b * _PCH, _PCH)], gsem.at[sl])
                for b in range(bsz)]

        def start_chunk(c):
            return [start_pos(c)] + start_gathers(c)

        pf = _NSLOT - 1
        inflight = {}
        outflight = {}

        pos_copies = {c: start_pos(c) for c in range(min(pf, nch))}
        idx_copies = [pltpu.async_copy(
            x_hbm.at[b, pl.ds(pbase, ppw)], idx_v.at[b], isem)
            for b in range(bsz)]
        for copy in idx_copies:
            copy.wait()
        for c in range(min(pf, nch)):
            inflight[c] = [pos_copies[c]] + start_gathers(c)

        for c in range(nch):
            sl = c % _NSLOT
            for copy in inflight.pop(c):
                copy.wait()

            def body(t, carry):
                p = t // 4
                jq = t % 4
                for g in range(ngrp // 4):
                    colw = (jq * (ngrp // 4) + g) * _LANES
                    col = colw * 4
                    pw = pbuf_v[sl, p, pl.ds(colw, _LANES)]
                    for k in range(4):
                        shl = pw << (24 - 8 * k) if k < 3 else pw
                        pv = lax.convert_element_type(
                            lax.shift_right_arithmetic(shl, 24),
                            jnp.float32) * dq
                        ck = col + k * _LANES
                        for b in range(bsz):
                            r = b * _PCH + p
                            v = rows_v[sl, r, pl.ds(ck, _LANES)] * scale + pv
                            rows_v[sl, r, pl.ds(ck, _LANES)] = v
                return carry

            lax.fori_loop(0, _PCH * 4, body, 0)

            wcopies = []
            for b in range(bsz):
                wcopies.append(pltpu.async_copy(
                    rows_v.at[sl, pl.ds(b * _PCH, _PCH)],
                    out_hbm.at[pl.ds(b * seq_len + pbase + c * _PCH, _PCH)],
                    osem.at[sl]))
            outflight[c] = wcopies

            nxt = c + pf
            if nxt < nch:
                prev = nxt - _NSLOT
                if prev >= 0:
                    for copy in outflight.pop(prev):
                        copy.wait()
                inflight[nxt] = start_chunk(nxt)

        for c in sorted(outflight):
            for copy in outflight.pop(c):
                copy.wait()

    return emb(x.astype(jnp.int32), table, pos)


def kernel(x, table):
    b, s = x.shape
    vocab, d = table.shape
    pos = jnp.asarray(_pos_encoding_packed_np(s, d))
    out = _run(x, table, pos)
    return out.reshape(b, s, d)
